# baseline (speedup 1.0000x reference)
import functools

import jax
import jax.numpy as jnp
from jax import lax
from jax.experimental import pallas as pl
from jax.experimental.pallas import tpu as pltpu
from jax.experimental.pallas import tpu_sc as plsc

B = 16384
POS_DIM = 42
SIDE_DIM = 22
TOTAL = 64
N_SIDE = 100
NC = 2
NS = 16
NW = NC * NS
BPW = B // NW
CH = 16
NCHUNK = BPW // CH
SPAD = 128


@jax.jit
def _sc_embed(item2d, side2d, pos_table, side_pad):
    mesh = plsc.VectorSubcoreMesh(core_axis_name="c", subcore_axis_name="s")

    @functools.partial(
        pl.kernel,
        out_type=jax.ShapeDtypeStruct((B, TOTAL), jnp.float32),
        mesh=mesh,
        scratch_types=[
            pltpu.VMEM((1, BPW), jnp.int32),
            pltpu.VMEM((1, BPW), jnp.int32),
            pltpu.VMEM((SPAD, SPAD), jnp.float32),
            pltpu.VMEM((CH, POS_DIM), jnp.float32),
            pltpu.VMEM((CH, POS_DIM), jnp.float32),
            pltpu.VMEM((BPW, TOTAL), jnp.float32),
            pltpu.SemaphoreType.DMA,
            pltpu.SemaphoreType.DMA,
        ],
    )
    def k(item_hbm, side_hbm, pos_hbm, sidet_hbm, out_hbm,
          iidx, sidx, side_v, pvA, pvB, comb, semA, semB):
        wid = lax.axis_index("s") * NC + lax.axis_index("c")
        base = wid * BPW
        pltpu.sync_copy(item_hbm.at[pl.ds(wid, 1)], iidx)
        pltpu.sync_copy(side_hbm.at[pl.ds(wid, 1)], sidx)
        pltpu.sync_copy(sidet_hbm, side_v)

        bufs = (pvA, pvB)
        sems = (semA, semB)

        def fire(c, buf, sem):
            ivec = iidx[0, pl.ds(c * CH, CH)]
            for u in range(CH):
                pltpu.async_copy(
                    pos_hbm.at[pl.ds(ivec[u], 1)], buf.at[pl.ds(u, 1)], sem)

        def drain(buf, sem):
            pltpu.make_async_copy(pos_hbm.at[pl.ds(0, CH)], buf, sem).wait()

        def assemble(c, buf):
            kbase = c * CH
            svec = sidx[0, pl.ds(kbase, CH)]
            for u in range(CH):
                kk = kbase + u
                s = svec[u]
                comb[kk, pl.ds(0, 16)] = buf[u, pl.ds(0, 16)]
                comb[kk, pl.ds(16, 16)] = buf[u, pl.ds(16, 16)]
                comb[kk, pl.ds(26, 16)] = buf[u, pl.ds(26, 16)]
                comb[kk, pl.ds(POS_DIM, 16)] = side_v[s, pl.ds(0, 16)]
                comb[kk, pl.ds(POS_DIM + 6, 16)] = side_v[s, pl.ds(6, 16)]

        fire(0, pvA, semA)
        fire(1, pvB, semB)

        def body(j, _):
            for p in range(2):
                c = j * 2 + p
                drain(bufs[p], sems[p])
                assemble(c, bufs[p])

                @pl.when(c + 2 < NCHUNK)
                def _():
                    fire(c + 2, bufs[p], sems[p])
            return ()

        lax.fori_loop(0, NCHUNK // 2, body, ())
        pltpu.sync_copy(comb, out_hbm.at[pl.ds(base, BPW)])

    return k(item2d, side2d, pos_table, side_pad)


def kernel(item_idx, side_idx, pos_table, side_table):
    item2d = item_idx.reshape(NW, BPW)
    side2d = side_idx.reshape(NW, BPW)
    side_pad = jnp.zeros((SPAD, SPAD), jnp.float32)
    side_pad = lax.dynamic_update_slice(side_pad, side_table, (0, 0))
    return _sc_embed(item2d, side2d, pos_table, side_pad)


# speedup vs baseline: 1.0009x; 1.0009x over previous
import functools

import jax
import jax.numpy as jnp
from jax import lax
from jax.experimental import pallas as pl
from jax.experimental.pallas import tpu as pltpu
from jax.experimental.pallas import tpu_sc as plsc

B = 16384
POS_DIM = 42
SIDE_DIM = 22
TOTAL = 64
N_SIDE = 100
NC = 2
NS = 16
NW = NC * NS
BPW = B // NW
CH = 16
NCHUNK = BPW // CH
SPAD = 128


@jax.jit
def _sc_embed(item2d, side2d, pos_table, side_pad):
    mesh = plsc.VectorSubcoreMesh(core_axis_name="c", subcore_axis_name="s")

    @functools.partial(
        pl.kernel,
        out_type=jax.ShapeDtypeStruct((B, TOTAL), jnp.float32),
        mesh=mesh,
        compiler_params=pltpu.CompilerParams(use_tc_tiling_on_sc=True),
        scratch_types=[
            pltpu.VMEM((1, BPW), jnp.int32),
            pltpu.VMEM((1, BPW), jnp.int32),
            pltpu.VMEM((SPAD, SPAD), jnp.float32),
            pltpu.VMEM((CH, POS_DIM), jnp.float32),
            pltpu.VMEM((CH, POS_DIM), jnp.float32),
            pltpu.VMEM((BPW, TOTAL), jnp.float32),
            pltpu.SemaphoreType.DMA,
            pltpu.SemaphoreType.DMA,
        ],
    )
    def k(item_hbm, side_hbm, pos_hbm, sidet_hbm, out_hbm,
          iidx, sidx, side_v, pvA, pvB, comb, semA, semB):
        wid = lax.axis_index("s") * NC + lax.axis_index("c")
        base = wid * BPW
        pltpu.sync_copy(item_hbm.at[pl.ds(wid, 1)], iidx)
        pltpu.sync_copy(side_hbm.at[pl.ds(wid, 1)], sidx)
        pltpu.sync_copy(sidet_hbm, side_v)

        bufs = (pvA, pvB)
        sems = (semA, semB)

        def fire(c, buf, sem):
            ivec = iidx[0, pl.ds(c * CH, CH)]
            for u in range(CH):
                pltpu.async_copy(
                    pos_hbm.at[pl.ds(ivec[u], 1)], buf.at[pl.ds(u, 1)], sem)

        def drain(buf, sem):
            pltpu.make_async_copy(pos_hbm.at[pl.ds(0, CH)], buf, sem).wait()

        def assemble(c, buf):
            kbase = c * CH
            svec = sidx[0, pl.ds(kbase, CH)]
            for u in range(CH):
                kk = kbase + u
                s = svec[u]
                comb[kk, pl.ds(0, 16)] = buf[u, pl.ds(0, 16)]
                comb[kk, pl.ds(16, 16)] = buf[u, pl.ds(16, 16)]
                comb[kk, pl.ds(26, 16)] = buf[u, pl.ds(26, 16)]
                comb[kk, pl.ds(POS_DIM, 16)] = side_v[s, pl.ds(0, 16)]
                comb[kk, pl.ds(POS_DIM + 6, 16)] = side_v[s, pl.ds(6, 16)]

        fire(0, pvA, semA)
        fire(1, pvB, semB)

        def body(j, _):
            for p in range(2):
                c = j * 2 + p
                drain(bufs[p], sems[p])
                assemble(c, bufs[p])

                @pl.when(c + 2 < NCHUNK)
                def _():
                    fire(c + 2, bufs[p], sems[p])
            return ()

        lax.fori_loop(0, NCHUNK // 2, body, ())
        pltpu.sync_copy(comb, out_hbm.at[pl.ds(base, BPW)])

    return k(item2d, side2d, pos_table, side_pad)


def kernel(item_idx, side_idx, pos_table, side_table):
    item2d = item_idx.reshape(NW, BPW)
    side2d = side_idx.reshape(NW, BPW)
    side_pad = jnp.zeros((SPAD, SPAD), jnp.float32)
    side_pad = lax.dynamic_update_slice(side_pad, side_table, (0, 0))
    return _sc_embed(item2d, side2d, pos_table, side_pad)


# speedup vs baseline: 1.0091x; 1.0082x over previous
import functools

import jax
import jax.numpy as jnp
from jax import lax
from jax.experimental import pallas as pl
from jax.experimental.pallas import tpu as pltpu
from jax.experimental.pallas import tpu_sc as plsc

B = 16384
POS_DIM = 42
SIDE_DIM = 22
TOTAL = 64
N_SIDE = 100
NC = 2
NS = 16
NW = NC * NS
BPW = B // NW
CH = 16
NCHUNK = BPW // CH
SPAD = 128


@jax.jit
def _sc_embed(item_idx, side_idx, pos_table, side_pad):
    mesh = plsc.VectorSubcoreMesh(core_axis_name="c", subcore_axis_name="s")

    @functools.partial(
        pl.kernel,
        out_type=jax.ShapeDtypeStruct((B, TOTAL), jnp.float32),
        mesh=mesh,
        compiler_params=pltpu.CompilerParams(use_tc_tiling_on_sc=True),
        scratch_types=[
            pltpu.VMEM((BPW,), jnp.int32),
            pltpu.VMEM((BPW,), jnp.int32),
            pltpu.VMEM((SPAD, SPAD), jnp.float32),
            pltpu.VMEM((CH, POS_DIM), jnp.float32),
            pltpu.VMEM((CH, POS_DIM), jnp.float32),
            pltpu.VMEM((BPW, TOTAL), jnp.float32),
            pltpu.SemaphoreType.DMA,
            pltpu.SemaphoreType.DMA,
        ],
    )
    def k(item_hbm, side_hbm, pos_hbm, sidet_hbm, out_hbm,
          iidx, sidx, side_v, pvA, pvB, comb, semA, semB):
        wid = lax.axis_index("s") * NC + lax.axis_index("c")
        base = wid * BPW
        pltpu.sync_copy(item_hbm.at[pl.ds(base, BPW)], iidx)
        pltpu.sync_copy(side_hbm.at[pl.ds(base, BPW)], sidx)
        pltpu.sync_copy(sidet_hbm, side_v)

        bufs = (pvA, pvB)
        sems = (semA, semB)

        def fire(c, buf, sem):
            ivec = iidx[pl.ds(c * CH, CH)]
            for u in range(CH):
                pltpu.async_copy(
                    pos_hbm.at[pl.ds(ivec[u], 1)], buf.at[pl.ds(u, 1)], sem)

        def drain(buf, sem):
            pltpu.make_async_copy(pos_hbm.at[pl.ds(0, CH)], buf, sem).wait()

        def assemble(c, buf):
            kbase = c * CH
            svec = sidx[pl.ds(kbase, CH)]
            for u in range(CH):
                kk = kbase + u
                s = svec[u]
                comb[kk, pl.ds(0, 16)] = buf[u, pl.ds(0, 16)]
                comb[kk, pl.ds(16, 16)] = buf[u, pl.ds(16, 16)]
                comb[kk, pl.ds(26, 16)] = buf[u, pl.ds(26, 16)]
                comb[kk, pl.ds(POS_DIM, 16)] = side_v[s, pl.ds(0, 16)]
                comb[kk, pl.ds(POS_DIM + 6, 16)] = side_v[s, pl.ds(6, 16)]

        fire(0, pvA, semA)
        fire(1, pvB, semB)

        def body(j, _):
            for p in range(2):
                c = j * 2 + p
                drain(bufs[p], sems[p])
                assemble(c, bufs[p])

                @pl.when(c + 2 < NCHUNK)
                def _():
                    fire(c + 2, bufs[p], sems[p])
            return ()

        lax.fori_loop(0, NCHUNK // 2, body, ())
        pltpu.sync_copy(comb, out_hbm.at[pl.ds(base, BPW)])

    return k(item_idx, side_idx, pos_table, side_pad)


def kernel(item_idx, side_idx, pos_table, side_table):
    side_pad = jnp.zeros((SPAD, SPAD), jnp.float32)
    side_pad = lax.dynamic_update_slice(side_pad, side_table, (0, 0))
    return _sc_embed(item_idx, side_idx, pos_table, side_pad)
